# baseline (speedup 1.0000x reference)
import functools

import jax
import jax.numpy as jnp
import numpy as np
from jax.experimental import pallas as pl
from jax.experimental.pallas import tpu as pltpu

_RATE = 0.5


@functools.lru_cache(maxsize=None)
def _feat_indices(b, hw, num):
    with jax.ensure_compile_time_eval():
        keys = jax.random.split(jax.random.key(42), b)
        fi = jax.vmap(lambda kk: jax.random.permutation(kk, hw)[:num])(keys)
        return np.asarray(fi)


def _lfm_kernel(sims_ref, fidx_ref, feat_ref, refs_ref, out_ref, ridx_ref):
    c, hw = feat_ref.shape[1], feat_ref.shape[2]
    num = fidx_ref.shape[2]
    n = ridx_ref.shape[1]
    bi = pl.program_id(0)

    feat = feat_ref[0]
    fidx = fidx_ref[0]

    iota = jax.lax.broadcasted_iota(jnp.int32, (hw, num), 0)
    oh_feat = (iota == fidx).astype(jnp.bfloat16)

    f_hi = feat.astype(jnp.bfloat16)
    r1 = feat - f_hi.astype(jnp.float32)
    f_mid = r1.astype(jnp.bfloat16)
    f_lo = (r1 - f_mid.astype(jnp.float32)).astype(jnp.bfloat16)
    dims_g = (((1,), (0,)), ((), ()))
    feat_sel = (jax.lax.dot_general(f_lo, oh_feat, dims_g,
                                    preferred_element_type=jnp.float32)
                + jax.lax.dot_general(f_mid, oh_feat, dims_g,
                                      preferred_element_type=jnp.float32)
                + jax.lax.dot_general(f_hi, oh_feat, dims_g,
                                      preferred_element_type=jnp.float32))
    norm_sel = jnp.maximum(
        jnp.sqrt(jnp.sum(feat_sel * feat_sel, axis=0, keepdims=True)), 1e-12)
    w1 = feat_sel / norm_sel
    n2 = jnp.maximum(jnp.sqrt(jnp.sum(w1 * w1, axis=0, keepdims=True)), 1e-12)
    wfs = w1 / n2
    wfs_bf16 = wfs.astype(jnp.bfloat16)

    base_sim = sims_ref[bi, 0]
    fused = base_sim * feat_sel

    for j in range(n):
        ref = refs_ref[0, j]
        rnorm = jnp.maximum(jnp.sqrt(jnp.sum(ref * ref, axis=0, keepdims=True)), 1e-12)
        wref = ref / rnorm
        fxT = jax.lax.dot_general(wref.astype(jnp.bfloat16), wfs_bf16,
                                  (((0,), (0,)), ((), ())),
                                  preferred_element_type=jnp.float32)
        cmax = jnp.max(fxT, axis=0, keepdims=True)
        amin = jnp.min(jnp.where(fxT == cmax, iota, hw), axis=0, keepdims=True)
        ridx_ref[0, j:j + 1, :] = amin
        oh = (iota == amin).astype(jnp.bfloat16)
        sj = sims_ref[bi, 1 + j]
        fused = fused + sj * jax.lax.dot_general(
            ref.astype(jnp.bfloat16), oh, dims_g,
            preferred_element_type=jnp.float32)

    scat = jax.lax.dot_general(fused.astype(jnp.bfloat16), oh_feat,
                               (((1,), (1,)), ((), ())),
                               preferred_element_type=jnp.float32)
    sel_mask = jax.lax.dot_general(jnp.ones((1, num), jnp.bfloat16), oh_feat,
                                   (((1,), (1,)), ((), ())),
                                   preferred_element_type=jnp.float32)
    out_ref[0] = jnp.where(sel_mask > 0.5, scat, feat)


def kernel(feat, refs, index, similarity):
    b, k, c, h, w = refs.shape
    hw = h * w
    n = k - 1
    num = int(_RATE * hw)

    try:
        feat_indices = jnp.asarray(_feat_indices(b, hw, num))
    except Exception:
        keys = jax.random.split(jax.random.key(42), b)
        feat_indices = jax.vmap(
            lambda kk: jax.random.permutation(kk, hw)[:num])(keys)

    feat3 = feat.reshape(b, c, hw)
    idx32 = jnp.asarray(index, jnp.int32)
    pos = jnp.arange(n, dtype=jnp.int32)
    sel = jnp.where(pos < idx32, pos, pos + 1)
    refs4 = jnp.take(refs, sel, axis=1).reshape(b, n, c, hw)
    sims_sel = jnp.concatenate(
        [jnp.take(similarity, idx32[None], axis=1),
         jnp.take(similarity, sel, axis=1)], axis=1).astype(jnp.float32)
    fidx3 = feat_indices.astype(jnp.int32).reshape(b, 1, num)

    out3, ridx = pl.pallas_call(
        _lfm_kernel,
        grid=(b,),
        in_specs=[
            pl.BlockSpec(memory_space=pltpu.SMEM),
            pl.BlockSpec((1, 1, num), lambda i: (i, 0, 0)),
            pl.BlockSpec((1, c, hw), lambda i: (i, 0, 0)),
            pl.BlockSpec((1, n, c, hw), lambda i: (i, 0, 0, 0)),
        ],
        out_specs=[
            pl.BlockSpec((1, c, hw), lambda i: (i, 0, 0)),
            pl.BlockSpec((1, n, num), lambda i: (i, 0, 0)),
        ],
        out_shape=[
            jax.ShapeDtypeStruct((b, c, hw), jnp.float32),
            jax.ShapeDtypeStruct((b, n, num), jnp.int32),
        ],
        compiler_params=pltpu.CompilerParams(
            dimension_semantics=("arbitrary",),
        ),
    )(sims_sel, fidx3, feat3, refs4)

    return out3.reshape(b, c, h, w), feat_indices, ridx


# speedup vs baseline: 3.4177x; 3.4177x over previous
import functools

import jax
import jax.numpy as jnp
import numpy as np
from jax.experimental import pallas as pl
from jax.experimental.pallas import tpu as pltpu

_RATE = 0.5


@functools.lru_cache(maxsize=None)
def _feat_indices(b, hw, num):
    with jax.ensure_compile_time_eval():
        keys = jax.random.split(jax.random.key(42), b)
        fi = jax.vmap(lambda kk: jax.random.permutation(kk, hw)[:num])(keys)
        return np.asarray(fi)


def _lfm_kernel(idx_ref, sims_ref, fidx_ref, feat_ref, refs_ref, out_ref, ridx_ref):
    c, hw = feat_ref.shape[1], feat_ref.shape[2]
    num = fidx_ref.shape[2]
    n = ridx_ref.shape[1]
    bi = pl.program_id(0)
    index = idx_ref[0, 0]

    feat = feat_ref[0]
    fidx = fidx_ref[0]

    iota = jax.lax.broadcasted_iota(jnp.int32, (hw, num), 0)
    oh_feat = (iota == fidx).astype(jnp.bfloat16)

    f_hi = feat.astype(jnp.bfloat16)
    r1 = feat - f_hi.astype(jnp.float32)
    f_mid = r1.astype(jnp.bfloat16)
    f_lo = (r1 - f_mid.astype(jnp.float32)).astype(jnp.bfloat16)
    dims_g = (((1,), (0,)), ((), ()))
    feat_sel = (jax.lax.dot_general(f_lo, oh_feat, dims_g,
                                    preferred_element_type=jnp.float32)
                + jax.lax.dot_general(f_mid, oh_feat, dims_g,
                                      preferred_element_type=jnp.float32)
                + jax.lax.dot_general(f_hi, oh_feat, dims_g,
                                      preferred_element_type=jnp.float32))
    norm_sel = jnp.maximum(
        jnp.sqrt(jnp.sum(feat_sel * feat_sel, axis=0, keepdims=True)), 1e-12)
    w1 = feat_sel / norm_sel
    n2 = jnp.maximum(jnp.sqrt(jnp.sum(w1 * w1, axis=0, keepdims=True)), 1e-12)
    wfs = w1 / n2
    wfs_bf16 = wfs.astype(jnp.bfloat16)

    base_sim = sims_ref[bi, index]
    fused = base_sim * feat_sel

    for j in range(n):
        jj = j + jnp.where(j >= index, 1, 0)
        ref = refs_ref[0, jj]
        rnorm = jnp.maximum(jnp.sqrt(jnp.sum(ref * ref, axis=0, keepdims=True)), 1e-12)
        wref = ref / rnorm
        fxT = jax.lax.dot_general(wref.astype(jnp.bfloat16), wfs_bf16,
                                  (((0,), (0,)), ((), ())),
                                  preferred_element_type=jnp.float32)
        cmax = jnp.max(fxT, axis=0, keepdims=True)
        amin = jnp.min(jnp.where(fxT == cmax, iota, hw), axis=0, keepdims=True)
        ridx_ref[0, j:j + 1, :] = amin
        oh = (iota == amin).astype(jnp.bfloat16)
        sj = sims_ref[bi, jj]
        fused = fused + sj * jax.lax.dot_general(
            ref.astype(jnp.bfloat16), oh, dims_g,
            preferred_element_type=jnp.float32)

    scat = jax.lax.dot_general(fused.astype(jnp.bfloat16), oh_feat,
                               (((1,), (1,)), ((), ())),
                               preferred_element_type=jnp.float32)
    sel_mask = jax.lax.dot_general(jnp.ones((1, num), jnp.bfloat16), oh_feat,
                                   (((1,), (1,)), ((), ())),
                                   preferred_element_type=jnp.float32)
    out_ref[0] = jnp.where(sel_mask > 0.5, scat, feat)


def kernel(feat, refs, index, similarity):
    b, k, c, h, w = refs.shape
    hw = h * w
    n = k - 1
    num = int(_RATE * hw)

    try:
        feat_indices = jnp.asarray(_feat_indices(b, hw, num))
    except Exception:
        keys = jax.random.split(jax.random.key(42), b)
        feat_indices = jax.vmap(
            lambda kk: jax.random.permutation(kk, hw)[:num])(keys)

    feat3 = feat.reshape(b, c, hw)
    refs4 = refs.reshape(b, k, c, hw)
    idx_arr = jnp.asarray(index, jnp.int32).reshape(1, 1)
    sims = similarity.astype(jnp.float32)
    fidx3 = feat_indices.astype(jnp.int32).reshape(b, 1, num)

    out3, ridx = pl.pallas_call(
        _lfm_kernel,
        grid=(b,),
        in_specs=[
            pl.BlockSpec(memory_space=pltpu.SMEM),
            pl.BlockSpec(memory_space=pltpu.SMEM),
            pl.BlockSpec((1, 1, num), lambda i: (i, 0, 0)),
            pl.BlockSpec((1, c, hw), lambda i: (i, 0, 0)),
            pl.BlockSpec((1, k, c, hw), lambda i: (i, 0, 0, 0)),
        ],
        out_specs=[
            pl.BlockSpec((1, c, hw), lambda i: (i, 0, 0)),
            pl.BlockSpec((1, n, num), lambda i: (i, 0, 0)),
        ],
        out_shape=[
            jax.ShapeDtypeStruct((b, c, hw), jnp.float32),
            jax.ShapeDtypeStruct((b, n, num), jnp.int32),
        ],
        compiler_params=pltpu.CompilerParams(
            dimension_semantics=("arbitrary",),
        ),
    )(idx_arr, sims, fidx3, feat3, refs4)

    return out3.reshape(b, c, h, w), feat_indices, ridx
